# Initial kernel scaffold; baseline (speedup 1.0000x reference)
import jax
import jax.numpy as jnp
from jax.experimental import pallas as pl


def kernel(x, edge_index, edge_attr, W_ne, b_ne, W_ee, b_ee, t, W1, b1, g1, bt1, W2, b2, ln_g, ln_b):
    raise NotImplementedError("write your pallas kernel here")


# speedup vs baseline: 2.0973x; 2.0973x over previous
import jax
import jax.numpy as jnp
from jax.experimental import pallas as pl

N = 10000
HID = 64
L = 14


def _ln(h, g, b, eps=1e-5):
    mu = jnp.mean(h, axis=-1, keepdims=True)
    var = jnp.var(h, axis=-1, keepdims=True)
    return (h - mu) * jax.lax.rsqrt(var + eps) * g + b


def _final_body(h_ref, g_ref, b_ref, o_ref):
    h = h_ref[...]
    o_ref[...] = jax.nn.relu(_ln(h, g_ref[...], b_ref[...]))


def _gen_conv(x, src, dst, ea, t, W1, b1, g1, bt1, W2, b2):
    msg = jax.nn.relu(x[src] + ea) + 1e-7
    ex = jnp.exp(msg * t)
    denom = jax.ops.segment_sum(ex, dst, num_segments=N)
    numer = jax.ops.segment_sum(ex * msg, dst, num_segments=N)
    aggr = numer / (denom + 1e-16)
    out = aggr + x
    h = out @ W1 + b1
    h = jax.nn.relu(_ln(h, g1, bt1))
    return h @ W2 + b2


def kernel(x, edge_index, edge_attr, W_ne, b_ne, W_ee, b_ee, t, W1, b1, g1, bt1, W2, b2, ln_g, ln_b):
    src = edge_index[0]
    dst = edge_index[1]
    h = x @ W_ne + b_ne
    ea = edge_attr @ W_ee + b_ee
    h = _gen_conv(h, src, dst, ea, t[0], W1[0], b1[0], g1[0], bt1[0], W2[0], b2[0])
    for i in range(1, L):
        r = jax.nn.relu(_ln(h, ln_g[i], ln_b[i]))
        r = _gen_conv(r, src, dst, ea, t[i], W1[i], b1[i], g1[i], bt1[i], W2[i], b2[i])
        h = h + r
    return pl.pallas_call(
        _final_body,
        out_shape=jax.ShapeDtypeStruct((N, HID), jnp.float32),
    )(h, ln_g[0], ln_b[0])


# speedup vs baseline: 8.5554x; 4.0792x over previous
import functools

import jax
import jax.numpy as jnp
from jax import lax
from jax.experimental import pallas as pl
from jax.experimental.pallas import tpu as pltpu
from jax.experimental.pallas import tpu_sc as plsc

N = 10000
E = 320000
D = 64
D2 = 128
L = 14

NC = 2
NS = 16
NW = NC * NS
EPT = E // NW
C = 80
NCH = EPT // C
ZR = 80
NCHR = N // ZR

_mesh = plsc.VectorSubcoreMesh(core_axis_name="c", subcore_axis_name="s")


def _edge_body(h_hbm, ea_hbm, src_hbm, dst_hbm, t_hbm,
               pn_hbm, pd_hbm,
               src_v, dst_v, rows_v, ea_v, sm_v, s_v, t_v, zero_v,
               accn, accd, sem):
    cid = lax.axis_index("c")
    sid = lax.axis_index("s")
    base = (cid * NS + sid) * EPT

    def _z(i, _):
        for j in range(4):
            zero_v[i, pl.ds(j * 16, 16)] = jnp.zeros((16,), jnp.float32)
        return 0
    lax.fori_loop(0, ZR, _z, 0)
    for k in range((NCHR + NS - 1) // NS):
        c = sid + k * NS

        @pl.when(c < NCHR)
        def _zero_chunk(c=c):
            pltpu.sync_copy(zero_v, accn.at[pl.ds(c * ZR, ZR)])
            pltpu.sync_copy(zero_v, accd.at[pl.ds(c * ZR, ZR)])
    pltpu.sync_copy(t_hbm, t_v)
    plsc.subcore_barrier()

    t = t_v[...]

    def _chunk(k, _):
        off = base + k * C
        pltpu.sync_copy(src_hbm.at[pl.ds(off, C)], src_v)
        pltpu.sync_copy(dst_hbm.at[pl.ds(off, C)], dst_v)
        pltpu.sync_copy(ea_hbm.at[pl.ds(off, C)], ea_v)
        pltpu.async_copy(h_hbm.at[src_v], rows_v, sem).wait()

        def _edge(e, _):
            for j in range(4):
                sl = pl.ds(j * 16, 16)
                m = jnp.maximum(rows_v[e, sl] + ea_v[e, sl], 0.0) + 1e-7
                s = jnp.exp(m * t)
                sm_v[e, sl] = s * m
                s_v[e, sl] = s
            return 0
        lax.fori_loop(0, C, _edge, 0)

        pltpu.sync_copy(sm_v, accn.at[dst_v], add=True)
        pltpu.sync_copy(s_v, accd.at[dst_v], add=True)
        return 0
    lax.fori_loop(0, NCH, _chunk, 0)

    plsc.subcore_barrier()
    for k in range((NCHR + NS - 1) // NS):
        c = sid + k * NS

        @pl.when(c < NCHR)
        def _read_chunk(c=c):
            pltpu.sync_copy(accn.at[pl.ds(c * ZR, ZR)], pn_hbm.at[cid, pl.ds(c * ZR, ZR)])
            pltpu.sync_copy(accd.at[pl.ds(c * ZR, ZR)], pd_hbm.at[cid, pl.ds(c * ZR, ZR)])


_edge_call = functools.partial(
    pl.kernel,
    out_type=[jax.ShapeDtypeStruct((NC, N, D), jnp.float32),
              jax.ShapeDtypeStruct((NC, N, D), jnp.float32)],
    mesh=_mesh,
    compiler_params=pltpu.CompilerParams(use_tc_tiling_on_sc=False),
    scratch_types=[
        pltpu.VMEM((C,), jnp.int32),
        pltpu.VMEM((C,), jnp.int32),
        pltpu.VMEM((C, D), jnp.float32),
        pltpu.VMEM((C, D), jnp.float32),
        pltpu.VMEM((C, D), jnp.float32),
        pltpu.VMEM((C, D), jnp.float32),
        pltpu.VMEM((16,), jnp.float32),
        pltpu.VMEM((ZR, D), jnp.float32),
        pltpu.VMEM_SHARED((N, D), jnp.float32),
        pltpu.VMEM_SHARED((N, D), jnp.float32),
        pltpu.SemaphoreType.DMA,
    ],
)(_edge_body)


def _ln(h, g, b, eps=1e-5):
    mu = jnp.mean(h, axis=-1, keepdims=True)
    var = jnp.var(h, axis=-1, keepdims=True)
    return (h - mu) * lax.rsqrt(var + eps) * g + b


NB = 1000


def _encode_body(x_ref, w_ref, b_ref, o_ref):
    o_ref[...] = jnp.dot(x_ref[...], w_ref[...],
                         preferred_element_type=jnp.float32) + b_ref[...]


def _encode(x, w, b):
    rows = x.shape[0]
    return pl.pallas_call(
        _encode_body,
        grid=(rows // NB,),
        in_specs=[
            pl.BlockSpec((NB, D2), lambda i: (i, 0)),
            pl.BlockSpec((D2, D), lambda i: (0, 0)),
            pl.BlockSpec((1, D), lambda i: (0, 0)),
        ],
        out_specs=pl.BlockSpec((NB, D), lambda i: (i, 0)),
        out_shape=jax.ShapeDtypeStruct((rows, D), jnp.float32),
    )(x, w, b)


def _node_body(pn_ref, pd_ref, cin_ref, hprev_ref,
               w1_ref, b1_ref, g1_ref, bt1_ref, w2_ref, b2_ref,
               gn_ref, bn_ref, hout_ref, nin_ref):
    numer = pn_ref[0] + pn_ref[1]
    denom = pd_ref[0] + pd_ref[1]
    out = numer / (denom + 1e-16) + cin_ref[...]
    hm = jnp.dot(out, w1_ref[...], preferred_element_type=jnp.float32) + b1_ref[...]
    hm = jax.nn.relu(_ln(hm, g1_ref[...], bt1_ref[...]))
    r = jnp.dot(hm, w2_ref[...], preferred_element_type=jnp.float32) + b2_ref[...]
    h = hprev_ref[...] + r
    hout_ref[...] = h
    nin_ref[...] = jax.nn.relu(_ln(h, gn_ref[...], bn_ref[...]))


def _node(pn, pd, cin, hprev, w1, b1, g1, bt1, w2, b2, gn, bn):
    vec = lambda: pl.BlockSpec((1, D2), lambda i: (0, 0))
    vec64 = lambda: pl.BlockSpec((1, D), lambda i: (0, 0))
    return pl.pallas_call(
        _node_body,
        grid=(N // NB,),
        in_specs=[
            pl.BlockSpec((NC, NB, D), lambda i: (0, i, 0)),
            pl.BlockSpec((NC, NB, D), lambda i: (0, i, 0)),
            pl.BlockSpec((NB, D), lambda i: (i, 0)),
            pl.BlockSpec((NB, D), lambda i: (i, 0)),
            pl.BlockSpec((D, D2), lambda i: (0, 0)),
            vec(), vec(), vec(),
            pl.BlockSpec((D2, D), lambda i: (0, 0)),
            vec64(), vec64(), vec64(),
        ],
        out_specs=[pl.BlockSpec((NB, D), lambda i: (i, 0)),
                   pl.BlockSpec((NB, D), lambda i: (i, 0))],
        out_shape=[jax.ShapeDtypeStruct((N, D), jnp.float32),
                   jax.ShapeDtypeStruct((N, D), jnp.float32)],
    )(pn, pd, cin, hprev, w1, b1, g1, bt1, w2, b2, gn, bn)


def kernel(x, edge_index, edge_attr, W_ne, b_ne, W_ee, b_ee, t, W1, b1, g1, bt1, W2, b2, ln_g, ln_b):
    src = edge_index[0]
    dst = edge_index[1]
    h0 = _encode(x, W_ne, b_ne.reshape(1, D))
    ea = _encode(edge_attr, W_ee, b_ee.reshape(1, D))
    tvecs = jnp.broadcast_to(t[:, None], (L, 16)).astype(jnp.float32)

    cin = h0
    hprev = jnp.zeros((N, D), jnp.float32)
    for i in range(L):
        pn, pd = _edge_call(cin, ea, src, dst, tvecs[i])
        j = (i + 1) % L
        hprev, cin = _node(pn, pd, cin, hprev,
                           W1[i], b1[i].reshape(1, D2), g1[i].reshape(1, D2),
                           bt1[i].reshape(1, D2), W2[i], b2[i].reshape(1, D),
                           ln_g[j].reshape(1, D), ln_b[j].reshape(1, D))
    return cin
